# Initial kernel scaffold; baseline (speedup 1.0000x reference)
import jax
import jax.numpy as jnp
from jax.experimental import pallas as pl


def kernel(query, key):
    raise NotImplementedError("write your pallas kernel here")


# speedup vs baseline: 3.0516x; 3.0516x over previous
import jax
import jax.numpy as jnp
from jax.experimental import pallas as pl

QK_D = 32
P2 = 256
TK = 16
BB = 4


def _body(q_ref, k_ref, w_ref, i_ref):
    scale = QK_D ** -0.5
    xs = []
    for b in range(BB):
        q = q_ref[b] * scale
        kk = k_ref[b]
        x = jax.lax.dot_general(q, kk, (((1,), (1,)), ((), ())),
                                preferred_element_type=jnp.float32)
        xs.append(x[None])
    x = jnp.concatenate(xs, axis=0)
    iota = jax.lax.broadcasted_iota(jnp.int32, (BB, P2, P2), 2)
    vals, idxs = [], []
    for _ in range(TK):
        m = jnp.max(x, axis=2, keepdims=True)
        idx = jnp.min(jnp.where(x == m, iota, P2), axis=2, keepdims=True)
        vals.append(m)
        idxs.append(idx)
        x = jnp.where(iota == idx, -jnp.inf, x)
    V = jnp.concatenate(vals, axis=2)
    I = jnp.concatenate(idxs, axis=2)
    E = jnp.exp(V - V[:, :, :1])
    s = jnp.sum(E, axis=2, keepdims=True)
    w_ref[...] = E / s
    i_ref[...] = I


def kernel(query, key):
    n = query.shape[0]
    w, i = pl.pallas_call(
        _body,
        grid=(n // BB,),
        in_specs=[
            pl.BlockSpec((BB, P2, QK_D), lambda g: (g, 0, 0)),
            pl.BlockSpec((BB, P2, QK_D), lambda g: (g, 0, 0)),
        ],
        out_specs=[
            pl.BlockSpec((BB, P2, TK), lambda g: (g, 0, 0)),
            pl.BlockSpec((BB, P2, TK), lambda g: (g, 0, 0)),
        ],
        out_shape=[
            jax.ShapeDtypeStruct((n, P2, TK), jnp.float32),
            jax.ShapeDtypeStruct((n, P2, TK), jnp.int32),
        ],
    )(query, key)
    return (w, i)


# speedup vs baseline: 4.3925x; 1.4394x over previous
import jax
import jax.numpy as jnp
from jax import lax
from jax.experimental import pallas as pl
from jax.experimental.pallas import tpu as pltpu
from jax.experimental.pallas import tpu_sc as plsc

QK_D = 32
P2 = 256
TK = 16
MB = 8
R = 64
NW = 32


def _mm_body(q_ref, k_ref, o_ref):
    scale = QK_D ** -0.5
    for b in range(MB):
        q = q_ref[b] * scale
        o_ref[b] = lax.dot_general(q, k_ref[b], (((1,), (1,)), ((), ())),
                                   preferred_element_type=jnp.float32)


def _logits(query, key):
    n = query.shape[0]
    return pl.pallas_call(
        _mm_body,
        grid=(n // MB,),
        in_specs=[
            pl.BlockSpec((MB, P2, QK_D), lambda g: (g, 0, 0)),
            pl.BlockSpec((MB, P2, QK_D), lambda g: (g, 0, 0)),
        ],
        out_specs=pl.BlockSpec((MB, P2, P2), lambda g: (g, 0, 0)),
        out_shape=jax.ShapeDtypeStruct((n, P2, P2), jnp.float32),
    )(query, key)


def _merge(av, ai, bv, bi):
    rbv = lax.rev(bv, (0,))
    rbi = lax.rev(bi, (0,))
    take = (av > rbv) | ((av == rbv) & (ai < rbi))
    mv = jnp.where(take, av, rbv)
    mi = jnp.where(take, ai, rbi)
    return plsc.sort_key_val(mv, mi, descending=True)


def _sc_body(n, lg, ow, oi, buf, wbuf, ibuf):
    c = lax.axis_index("c")
    s = lax.axis_index("s")
    wid = s * 2 + c
    bpw = n // NW
    cpb = P2 // R
    idx_consts = [lax.iota(jnp.int32, 16) + 16 * j for j in range(16)]

    def row_body(r, carry):
        pairs = []
        for j in range(16):
            v = buf[r, pl.ds(16 * j, 16)]
            pairs.append(plsc.sort_key_val(v, idx_consts[j], descending=True))
        while len(pairs) > 1:
            pairs = [_merge(*pairs[t], *pairs[t + 1])
                     for t in range(0, len(pairs), 2)]
        tv, ti = pairs[0]
        m = jnp.max(tv)
        e = jnp.exp(tv - m)
        wbuf[r] = e / jnp.sum(e)
        ibuf[r] = ti
        return carry

    def chunk_body(ci, carry):
        b = wid * bpw + ci // cpb
        r0 = (ci % cpb) * R
        pltpu.sync_copy(lg.at[b, pl.ds(r0, R)], buf)
        lax.fori_loop(0, R, row_body, 0)
        pltpu.sync_copy(wbuf, ow.at[b, pl.ds(r0, R)])
        pltpu.sync_copy(ibuf, oi.at[b, pl.ds(r0, R)])
        return carry

    lax.fori_loop(0, bpw * cpb, chunk_body, 0)


def _sc_topk(logits):
    n = logits.shape[0]
    mesh = plsc.VectorSubcoreMesh(core_axis_name="c", subcore_axis_name="s")
    f = pl.kernel(
        lambda *refs: _sc_body(n, *refs),
        out_type=[
            jax.ShapeDtypeStruct((n, P2, TK), jnp.float32),
            jax.ShapeDtypeStruct((n, P2, TK), jnp.int32),
        ],
        mesh=mesh,
        compiler_params=pltpu.CompilerParams(needs_layout_passes=False),
        scratch_types=[
            pltpu.VMEM((R, P2), jnp.float32),
            pltpu.VMEM((R, TK), jnp.float32),
            pltpu.VMEM((R, TK), jnp.int32),
        ],
    )
    return f(logits)


def kernel(query, key):
    lg = _logits(query, key)
    w, i = _sc_topk(lg)
    return (w, i)


# speedup vs baseline: 4.4881x; 1.0218x over previous
import jax
import jax.numpy as jnp
from jax import lax
from jax.experimental import pallas as pl
from jax.experimental.pallas import tpu as pltpu
from jax.experimental.pallas import tpu_sc as plsc

QK_D = 32
P2 = 256
TK = 16
MB = 8
R = 64
NW = 32


def _mm_body(q_ref, k_ref, o_ref):
    scale = QK_D ** -0.5
    for b in range(MB):
        q = q_ref[b] * scale
        o_ref[b] = lax.dot_general(q, k_ref[b], (((1,), (1,)), ((), ())),
                                   preferred_element_type=jnp.float32)


def _logits(query, key):
    n = query.shape[0]
    return pl.pallas_call(
        _mm_body,
        grid=(n // MB,),
        in_specs=[
            pl.BlockSpec((MB, P2, QK_D), lambda g: (g, 0, 0)),
            pl.BlockSpec((MB, P2, QK_D), lambda g: (g, 0, 0)),
        ],
        out_specs=pl.BlockSpec((MB, P2, P2), lambda g: (g, 0, 0)),
        out_shape=jax.ShapeDtypeStruct((n, P2, P2), jnp.float32),
    )(query, key)


def _merge(av, ai, bv, bi, descending):
    take = (av > bv) | ((av == bv) & (ai < bi))
    mv = jnp.where(take, av, bv)
    mi = jnp.where(take, ai, bi)
    return plsc.sort_key_val(mv, mi, descending=descending)


def _sc_body(n, lg, ow, oi, buf, wbuf, ibuf):
    c = lax.axis_index("c")
    s = lax.axis_index("s")
    wid = s * 2 + c
    bpw = n // NW
    cpb = P2 // R
    idx_consts = [lax.iota(jnp.int32, 16) + 16 * j for j in range(16)]

    def row_body(r, carry):
        pairs = []
        for j in range(16):
            v = buf[r, pl.ds(16 * j, 16)]
            pairs.append(plsc.sort_key_val(v, idx_consts[j],
                                           descending=(j % 2 == 0)))
        while len(pairs) > 1:
            pairs = [_merge(*pairs[t], *pairs[t + 1],
                            descending=((t // 2) % 2 == 0 or len(pairs) == 2))
                     for t in range(0, len(pairs), 2)]
        tv, ti = pairs[0]
        m = jnp.max(tv)
        e = jnp.exp(tv - m)
        wbuf[r] = e / jnp.sum(e)
        ibuf[r] = ti
        return carry

    def chunk_body(ci, carry):
        b = wid * bpw + ci // cpb
        r0 = (ci % cpb) * R
        pltpu.sync_copy(lg.at[b, pl.ds(r0, R)], buf)
        lax.fori_loop(0, R, row_body, 0)
        pltpu.sync_copy(wbuf, ow.at[b, pl.ds(r0, R)])
        pltpu.sync_copy(ibuf, oi.at[b, pl.ds(r0, R)])
        return carry

    lax.fori_loop(0, bpw * cpb, chunk_body, 0)


def _sc_topk(logits):
    n = logits.shape[0]
    mesh = plsc.VectorSubcoreMesh(core_axis_name="c", subcore_axis_name="s")
    f = pl.kernel(
        lambda *refs: _sc_body(n, *refs),
        out_type=[
            jax.ShapeDtypeStruct((n, P2, TK), jnp.float32),
            jax.ShapeDtypeStruct((n, P2, TK), jnp.int32),
        ],
        mesh=mesh,
        compiler_params=pltpu.CompilerParams(needs_layout_passes=False),
        scratch_types=[
            pltpu.VMEM((R, P2), jnp.float32),
            pltpu.VMEM((R, TK), jnp.float32),
            pltpu.VMEM((R, TK), jnp.int32),
        ],
    )
    return f(logits)


def kernel(query, key):
    lg = _logits(query, key)
    w, i = _sc_topk(lg)
    return (w, i)


# speedup vs baseline: 5.4325x; 1.2104x over previous
import jax
import jax.numpy as jnp
from jax import lax
from jax.experimental import pallas as pl
from jax.experimental.pallas import tpu as pltpu
from jax.experimental.pallas import tpu_sc as plsc

QK_D = 32
P2 = 256
TK = 16
MB = 8
R = 64
NW = 32


def _mm_body(q_ref, k_ref, o_ref):
    scale = QK_D ** -0.5
    for b in range(MB):
        q = q_ref[b] * scale
        o_ref[b] = lax.dot_general(q, k_ref[b], (((1,), (1,)), ((), ())),
                                   preferred_element_type=jnp.float32)


def _logits(query, key):
    n = query.shape[0]
    return pl.pallas_call(
        _mm_body,
        grid=(n // MB,),
        in_specs=[
            pl.BlockSpec((MB, P2, QK_D), lambda g: (g, 0, 0)),
            pl.BlockSpec((MB, P2, QK_D), lambda g: (g, 0, 0)),
        ],
        out_specs=pl.BlockSpec((MB, P2, P2), lambda g: (g, 0, 0)),
        out_shape=jax.ShapeDtypeStruct((n, P2, P2), jnp.float32),
    )(query, key)


def _merge(av, ai, bv, bi, descending):
    take = (av > bv) | ((av == bv) & (ai < bi))
    mv = jnp.where(take, av, bv)
    mi = jnp.where(take, ai, bi)
    return plsc.sort_key_val(mv, mi, descending=descending)


def _sc_body(n, lg, ow, oi, buf0, buf1, wb0, wb1, ib0, ib1,
             isem0, isem1, osem0, osem1):
    c = lax.axis_index("c")
    s = lax.axis_index("s")
    wid = s * 2 + c
    bpw = n // NW
    cpb = P2 // R
    nch = bpw * cpb
    idx_consts = [lax.iota(jnp.int32, 16) + 16 * j for j in range(16)]
    bufs = ((buf0, wb0, ib0, isem0, osem0), (buf1, wb1, ib1, isem1, osem1))

    def chunk_slices(ci):
        b = wid * bpw + ci // cpb
        r0 = (ci % cpb) * R
        return (lg.at[b, pl.ds(r0, R)],
                ow.at[b, pl.ds(r0, R)],
                oi.at[b, pl.ds(r0, R)])

    def make_row_body(buf, wbuf, ibuf):
        def row_body(r, carry):
            pairs = []
            for j in range(16):
                v = buf[r, pl.ds(16 * j, 16)]
                pairs.append(plsc.sort_key_val(v, idx_consts[j],
                                               descending=(j % 2 == 0)))
            while len(pairs) > 1:
                pairs = [_merge(*pairs[t], *pairs[t + 1],
                                descending=((t // 2) % 2 == 0
                                            or len(pairs) == 2))
                         for t in range(0, len(pairs), 2)]
            tv, ti = pairs[0]
            m = jnp.max(tv)
            e = jnp.exp(tv - m)
            wbuf[r] = e / jnp.sum(e)
            ibuf[r] = ti
            return carry
        return row_body

    for par in (0, 1):
        buf, _, _, isem, _ = bufs[par]
        src, _, _ = chunk_slices(par)
        pltpu.async_copy(src, buf, isem)

    def pair_body(i, carry):
        for par in (0, 1):
            buf, wbuf, ibuf, isem, osem = bufs[par]
            ci = 2 * i + par
            src, wdst, idst = chunk_slices(ci)
            pltpu.make_async_copy(src, buf, isem).wait()

            @pl.when(i > 0)
            def _():
                pltpu.make_async_copy(wbuf, wdst, osem).wait()
                pltpu.make_async_copy(ibuf, idst, osem).wait()

            lax.fori_loop(0, R, make_row_body(buf, wbuf, ibuf), 0)
            pltpu.async_copy(wbuf, wdst, osem)
            pltpu.async_copy(ibuf, idst, osem)
            nsrc, _, _ = chunk_slices((ci + 2) % nch)
            pltpu.async_copy(nsrc, buf, isem)
        return carry

    lax.fori_loop(0, nch // 2, pair_body, 0)

    for par in (0, 1):
        buf, wbuf, ibuf, isem, osem = bufs[par]
        src, wdst, idst = chunk_slices(par)
        pltpu.make_async_copy(src, buf, isem).wait()
        pltpu.make_async_copy(wbuf, wdst, osem).wait()
        pltpu.make_async_copy(ibuf, idst, osem).wait()


def _sc_topk(logits):
    n = logits.shape[0]
    mesh = plsc.VectorSubcoreMesh(core_axis_name="c", subcore_axis_name="s")
    f = pl.kernel(
        lambda *refs: _sc_body(n, *refs),
        out_type=[
            jax.ShapeDtypeStruct((n, P2, TK), jnp.float32),
            jax.ShapeDtypeStruct((n, P2, TK), jnp.int32),
        ],
        mesh=mesh,
        compiler_params=pltpu.CompilerParams(needs_layout_passes=False),
        scratch_types=[
            pltpu.VMEM((R, P2), jnp.float32),
            pltpu.VMEM((R, P2), jnp.float32),
            pltpu.VMEM((R, TK), jnp.float32),
            pltpu.VMEM((R, TK), jnp.float32),
            pltpu.VMEM((R, TK), jnp.int32),
            pltpu.VMEM((R, TK), jnp.int32),
            pltpu.SemaphoreType.DMA,
            pltpu.SemaphoreType.DMA,
            pltpu.SemaphoreType.DMA,
            pltpu.SemaphoreType.DMA,
        ],
    )
    return f(logits)


def kernel(query, key):
    lg = _logits(query, key)
    w, i = _sc_topk(lg)
    return (w, i)


# speedup vs baseline: 6.3769x; 1.1738x over previous
import jax
import jax.numpy as jnp
from jax import lax
from jax.experimental import pallas as pl
from jax.experimental.pallas import tpu as pltpu
from jax.experimental.pallas import tpu_sc as plsc

QK_D = 32
P2 = 256
TK = 16
MB = 8
R = 64
NW = 32


def _mm_body(q_ref, k_ref, o_ref):
    scale = QK_D ** -0.5
    for b in range(MB):
        q = q_ref[b] * scale
        o_ref[b] = lax.dot_general(q, k_ref[b], (((1,), (1,)), ((), ())),
                                   preferred_element_type=jnp.float32)


def _logits(query, key):
    n = query.shape[0]
    return pl.pallas_call(
        _mm_body,
        grid=(n // MB,),
        in_specs=[
            pl.BlockSpec((MB, P2, QK_D), lambda g: (g, 0, 0)),
            pl.BlockSpec((MB, P2, QK_D), lambda g: (g, 0, 0)),
        ],
        out_specs=pl.BlockSpec((MB, P2, P2), lambda g: (g, 0, 0)),
        out_shape=jax.ShapeDtypeStruct((n, P2, P2), jnp.float32),
    )(query, key)


def _merge(av, ai, bv, bi, descending):
    take = (av > bv) | ((av == bv) & (ai < bi))
    mv = jnp.where(take, av, bv)
    mi = jnp.where(take, ai, bi)
    return plsc.sort_key_val(mv, mi, descending=descending)


def _sc_body(n, lg, ow, oi, buf0, buf1, wb0, wb1, ib0, ib1,
             isem0, isem1, osem0, osem1):
    c = lax.axis_index("c")
    s = lax.axis_index("s")
    wid = s * 2 + c
    bpw = n // NW
    cpb = P2 // R
    nch = bpw * cpb
    idx_consts = [lax.iota(jnp.int32, 16) + 16 * j for j in range(16)]
    bufs = ((buf0, wb0, ib0, isem0, osem0), (buf1, wb1, ib1, isem1, osem1))

    def chunk_slices(ci):
        b = wid * bpw + ci // cpb
        r0 = (ci % cpb) * R
        return (lg.at[b, pl.ds(r0, R)],
                ow.at[b, pl.ds(r0, R)],
                oi.at[b, pl.ds(r0, R)])

    def make_row_body(buf, wbuf, ibuf):
        def row_body(r):
            pairs = []
            for j in range(16):
                v = buf[r, pl.ds(16 * j, 16)]
                pairs.append(plsc.sort_key_val(v, idx_consts[j],
                                               descending=(j % 2 == 0)))
            while len(pairs) > 1:
                pairs = [_merge(*pairs[t], *pairs[t + 1],
                                descending=((t // 2) % 2 == 0
                                            or len(pairs) == 2))
                         for t in range(0, len(pairs), 2)]
            tv, ti = pairs[0]
            m = jnp.max(tv)
            e = jnp.exp(tv - m)
            wbuf[r] = e / jnp.sum(e)
            ibuf[r] = ti
        return row_body

    for par in (0, 1):
        buf, _, _, isem, _ = bufs[par]
        src, _, _ = chunk_slices(par)
        pltpu.async_copy(src, buf, isem)

    def pair_body(i, carry):
        for par in (0, 1):
            buf, wbuf, ibuf, isem, osem = bufs[par]
            ci = 2 * i + par
            src, wdst, idst = chunk_slices(ci)
            pltpu.make_async_copy(src, buf, isem).wait()

            @pl.when(i > 0)
            def _():
                pltpu.make_async_copy(wbuf, wdst, osem).wait()
                pltpu.make_async_copy(ibuf, idst, osem).wait()

            plsc.parallel_loop(0, R, unroll=4)(make_row_body(buf, wbuf, ibuf))
            pltpu.async_copy(wbuf, wdst, osem)
            pltpu.async_copy(ibuf, idst, osem)
            nsrc, _, _ = chunk_slices((ci + 2) % nch)
            pltpu.async_copy(nsrc, buf, isem)
        return carry

    lax.fori_loop(0, nch // 2, pair_body, 0)

    for par in (0, 1):
        buf, wbuf, ibuf, isem, osem = bufs[par]
        src, wdst, idst = chunk_slices(par)
        pltpu.make_async_copy(src, buf, isem).wait()
        pltpu.make_async_copy(wbuf, wdst, osem).wait()
        pltpu.make_async_copy(ibuf, idst, osem).wait()


def _sc_topk(logits):
    n = logits.shape[0]
    mesh = plsc.VectorSubcoreMesh(core_axis_name="c", subcore_axis_name="s")
    f = pl.kernel(
        lambda *refs: _sc_body(n, *refs),
        out_type=[
            jax.ShapeDtypeStruct((n, P2, TK), jnp.float32),
            jax.ShapeDtypeStruct((n, P2, TK), jnp.int32),
        ],
        mesh=mesh,
        compiler_params=pltpu.CompilerParams(needs_layout_passes=False),
        scratch_types=[
            pltpu.VMEM((R, P2), jnp.float32),
            pltpu.VMEM((R, P2), jnp.float32),
            pltpu.VMEM((R, TK), jnp.float32),
            pltpu.VMEM((R, TK), jnp.float32),
            pltpu.VMEM((R, TK), jnp.int32),
            pltpu.VMEM((R, TK), jnp.int32),
            pltpu.SemaphoreType.DMA,
            pltpu.SemaphoreType.DMA,
            pltpu.SemaphoreType.DMA,
            pltpu.SemaphoreType.DMA,
        ],
    )
    return f(logits)


def kernel(query, key):
    lg = _logits(query, key)
    w, i = _sc_topk(lg)
    return (w, i)


# speedup vs baseline: 6.7947x; 1.0655x over previous
import jax
import jax.numpy as jnp
from jax import lax
from jax.experimental import pallas as pl
from jax.experimental.pallas import tpu as pltpu
from jax.experimental.pallas import tpu_sc as plsc

QK_D = 32
P2 = 256
TK = 16
MB = 8
R = 64
NW = 32


def _mm_body(q_ref, k_ref, o_ref):
    scale = QK_D ** -0.5
    for b in range(MB):
        q = q_ref[b] * scale
        o_ref[b] = lax.dot_general(q, k_ref[b], (((1,), (1,)), ((), ())),
                                   preferred_element_type=jnp.float32)


def _logits(query, key):
    n = query.shape[0]
    return pl.pallas_call(
        _mm_body,
        grid=(n // MB,),
        in_specs=[
            pl.BlockSpec((MB, P2, QK_D), lambda g: (g, 0, 0)),
            pl.BlockSpec((MB, P2, QK_D), lambda g: (g, 0, 0)),
        ],
        out_specs=pl.BlockSpec((MB, P2, P2), lambda g: (g, 0, 0)),
        out_shape=jax.ShapeDtypeStruct((n, P2, P2), jnp.float32),
    )(query, key)


def _merge(av, ai, bv, bi, descending):
    take = (av > bv) | ((av == bv) & (ai < bi))
    mv = jnp.where(take, av, bv)
    mi = jnp.where(take, ai, bi)
    return plsc.sort_key_val(mv, mi, descending=descending)


def _sc_body(n, lg, ow, oi, buf0, buf1, wb0, wb1, ib0, ib1,
             isem0, isem1, osem0, osem1):
    c = lax.axis_index("c")
    s = lax.axis_index("s")
    wid = s * 2 + c
    bpw = n // NW
    cpb = P2 // R
    nch = bpw * cpb
    idx_consts = [lax.iota(jnp.int32, 16) + 16 * j for j in range(16)]
    bufs = ((buf0, wb0, ib0, isem0, osem0), (buf1, wb1, ib1, isem1, osem1))

    def chunk_slices(ci):
        b = wid * bpw + ci // cpb
        r0 = (ci % cpb) * R
        return (lg.at[b, pl.ds(r0, R)],
                ow.at[b, pl.ds(r0, R)],
                oi.at[b, pl.ds(r0, R)])

    def make_row_body(buf, wbuf, ibuf):
        def row_body(r):
            pairs = []
            for j in range(16):
                v = buf[r, pl.ds(16 * j, 16)]
                pairs.append(plsc.sort_key_val(v, idx_consts[j],
                                               descending=(j % 2 == 0)))
            while len(pairs) > 1:
                pairs = [_merge(*pairs[t], *pairs[t + 1],
                                descending=((t // 2) % 2 == 0
                                            or len(pairs) == 2))
                         for t in range(0, len(pairs), 2)]
            tv, ti = pairs[0]
            m = jnp.max(tv)
            e = jnp.exp(tv - m)
            wbuf[r] = e / jnp.sum(e)
            ibuf[r] = ti
        return row_body

    for par in (0, 1):
        buf, _, _, isem, _ = bufs[par]
        src, _, _ = chunk_slices(par)
        pltpu.async_copy(src, buf, isem)

    def pair_body(i, carry):
        for par in (0, 1):
            buf, wbuf, ibuf, isem, osem = bufs[par]
            ci = 2 * i + par
            src, wdst, idst = chunk_slices(ci)
            pltpu.make_async_copy(src, buf, isem).wait()

            @pl.when(i > 0)
            def _():
                pltpu.make_async_copy(wbuf, wdst, osem).wait()
                pltpu.make_async_copy(ibuf, idst, osem).wait()

            plsc.parallel_loop(0, R, unroll=8)(make_row_body(buf, wbuf, ibuf))
            pltpu.async_copy(wbuf, wdst, osem)
            pltpu.async_copy(ibuf, idst, osem)
            nsrc, _, _ = chunk_slices((ci + 2) % nch)
            pltpu.async_copy(nsrc, buf, isem)
        return carry

    lax.fori_loop(0, nch // 2, pair_body, 0)

    for par in (0, 1):
        buf, wbuf, ibuf, isem, osem = bufs[par]
        src, wdst, idst = chunk_slices(par)
        pltpu.make_async_copy(src, buf, isem).wait()
        pltpu.make_async_copy(wbuf, wdst, osem).wait()
        pltpu.make_async_copy(ibuf, idst, osem).wait()


def _sc_topk(logits):
    n = logits.shape[0]
    mesh = plsc.VectorSubcoreMesh(core_axis_name="c", subcore_axis_name="s")
    f = pl.kernel(
        lambda *refs: _sc_body(n, *refs),
        out_type=[
            jax.ShapeDtypeStruct((n, P2, TK), jnp.float32),
            jax.ShapeDtypeStruct((n, P2, TK), jnp.int32),
        ],
        mesh=mesh,
        compiler_params=pltpu.CompilerParams(needs_layout_passes=False),
        scratch_types=[
            pltpu.VMEM((R, P2), jnp.float32),
            pltpu.VMEM((R, P2), jnp.float32),
            pltpu.VMEM((R, TK), jnp.float32),
            pltpu.VMEM((R, TK), jnp.float32),
            pltpu.VMEM((R, TK), jnp.int32),
            pltpu.VMEM((R, TK), jnp.int32),
            pltpu.SemaphoreType.DMA,
            pltpu.SemaphoreType.DMA,
            pltpu.SemaphoreType.DMA,
            pltpu.SemaphoreType.DMA,
        ],
    )
    return f(logits)


def kernel(query, key):
    lg = _logits(query, key)
    w, i = _sc_topk(lg)
    return (w, i)


# speedup vs baseline: 7.7447x; 1.1398x over previous
import jax
import jax.numpy as jnp
from jax import lax
from jax.experimental import pallas as pl
from jax.experimental.pallas import tpu as pltpu
from jax.experimental.pallas import tpu_sc as plsc

QK_D = 32
P2 = 256
TK = 16
MB = 8
R = 64
NW = 32


def _mm_body(q_ref, k_ref, o_ref):
    scale = QK_D ** -0.5
    for b in range(MB):
        q = q_ref[b] * scale
        o_ref[b] = lax.dot_general(q, k_ref[b], (((1,), (1,)), ((), ())),
                                   preferred_element_type=jnp.float32)


def _logits(query, key, g0, gs):
    return pl.pallas_call(
        _mm_body,
        grid=(gs // MB,),
        in_specs=[
            pl.BlockSpec((MB, P2, QK_D), lambda t, o=g0 // MB: (o + t, 0, 0)),
            pl.BlockSpec((MB, P2, QK_D), lambda t, o=g0 // MB: (o + t, 0, 0)),
        ],
        out_specs=pl.BlockSpec((MB, P2, P2), lambda t: (t, 0, 0)),
        out_shape=jax.ShapeDtypeStruct((gs, P2, P2), jnp.float32),
    )(query, key)


def _merge(av, ai, bv, bi, descending):
    take = (av > bv) | ((av == bv) & (ai < bi))
    mv = jnp.where(take, av, bv)
    mi = jnp.where(take, ai, bi)
    return plsc.sort_key_val(mv, mi, descending=descending)


def _sc_body(n, lg, ow, oi, buf0, buf1, wb0, wb1, ib0, ib1,
             isem0, isem1, osem0, osem1):
    c = lax.axis_index("c")
    s = lax.axis_index("s")
    wid = s * 2 + c
    bpw = n // NW
    cpb = P2 // R
    nch = bpw * cpb
    idx_consts = [lax.iota(jnp.int32, 16) + 16 * j for j in range(16)]
    bufs = ((buf0, wb0, ib0, isem0, osem0), (buf1, wb1, ib1, isem1, osem1))

    def chunk_slices(ci):
        b = wid * bpw + ci // cpb
        r0 = (ci % cpb) * R
        return (lg.at[b, pl.ds(r0, R)],
                ow.at[b, pl.ds(r0, R)],
                oi.at[b, pl.ds(r0, R)])

    def make_row_body(buf, wbuf, ibuf):
        def row_body(r):
            pairs = []
            for j in range(16):
                v = buf[r, pl.ds(16 * j, 16)]
                pairs.append(plsc.sort_key_val(v, idx_consts[j],
                                               descending=(j % 2 == 0)))
            while len(pairs) > 1:
                pairs = [_merge(*pairs[t], *pairs[t + 1],
                                descending=((t // 2) % 2 == 0
                                            or len(pairs) == 2))
                         for t in range(0, len(pairs), 2)]
            tv, ti = pairs[0]
            m = jnp.max(tv)
            e = jnp.exp(tv - m)
            wbuf[r] = e / jnp.sum(e)
            ibuf[r] = ti
        return row_body

    for par in (0, 1):
        buf, _, _, isem, _ = bufs[par]
        src, _, _ = chunk_slices(par)
        pltpu.async_copy(src, buf, isem)

    def pair_body(i, carry):
        for par in (0, 1):
            buf, wbuf, ibuf, isem, osem = bufs[par]
            ci = 2 * i + par
            src, wdst, idst = chunk_slices(ci)
            pltpu.make_async_copy(src, buf, isem).wait()

            @pl.when(i > 0)
            def _():
                pltpu.make_async_copy(wbuf, wdst, osem).wait()
                pltpu.make_async_copy(ibuf, idst, osem).wait()

            plsc.parallel_loop(0, R, unroll=8)(make_row_body(buf, wbuf, ibuf))
            pltpu.async_copy(wbuf, wdst, osem)
            pltpu.async_copy(ibuf, idst, osem)
            nsrc, _, _ = chunk_slices((ci + 2) % nch)
            pltpu.async_copy(nsrc, buf, isem)
        return carry

    lax.fori_loop(0, nch // 2, pair_body, 0)

    for par in (0, 1):
        buf, wbuf, ibuf, isem, osem = bufs[par]
        src, wdst, idst = chunk_slices(par)
        pltpu.make_async_copy(src, buf, isem).wait()
        pltpu.make_async_copy(wbuf, wdst, osem).wait()
        pltpu.make_async_copy(ibuf, idst, osem).wait()


def _sc_topk(logits):
    n = logits.shape[0]
    mesh = plsc.VectorSubcoreMesh(core_axis_name="c", subcore_axis_name="s")
    f = pl.kernel(
        lambda *refs: _sc_body(n, *refs),
        out_type=[
            jax.ShapeDtypeStruct((n, P2, TK), jnp.float32),
            jax.ShapeDtypeStruct((n, P2, TK), jnp.int32),
        ],
        mesh=mesh,
        compiler_params=pltpu.CompilerParams(needs_layout_passes=False),
        scratch_types=[
            pltpu.VMEM((R, P2), jnp.float32),
            pltpu.VMEM((R, P2), jnp.float32),
            pltpu.VMEM((R, TK), jnp.float32),
            pltpu.VMEM((R, TK), jnp.float32),
            pltpu.VMEM((R, TK), jnp.int32),
            pltpu.VMEM((R, TK), jnp.int32),
            pltpu.SemaphoreType.DMA,
            pltpu.SemaphoreType.DMA,
            pltpu.SemaphoreType.DMA,
            pltpu.SemaphoreType.DMA,
        ],
    )
    return f(logits)


GROUPS = 4


def kernel(query, key):
    n = query.shape[0]
    gs = n // GROUPS
    ws, idxs = [], []
    for g in range(GROUPS):
        lg = _logits(query, key, g * gs, gs)
        w, i = _sc_topk(lg)
        ws.append(w)
        idxs.append(i)
    return (jnp.concatenate(ws), jnp.concatenate(idxs))
